# Initial kernel scaffold; baseline (speedup 1.0000x reference)
import jax
import jax.numpy as jnp
from jax.experimental import pallas as pl


def kernel(inputs, embedding):
    raise NotImplementedError("write your pallas kernel here")


# speedup vs baseline: 1.2569x; 1.2569x over previous
import functools

import jax
import jax.numpy as jnp
from jax import lax
from jax.experimental import pallas as pl
from jax.experimental.pallas import tpu as pltpu
from jax.experimental.pallas import tpu_sc as plsc

NE = 8192
ED = 256
N = 8192
TI = 256
TJ = 2048


def _argmin_body(x_ref, embt_ref, x2_ref, e2_ref, idx_ref):
    x = x_ref[...]
    x2 = x2_ref[...]
    mv = None
    mi = None
    for j in range(NE // TJ):
        et = embt_ref[:, j * TJ:(j + 1) * TJ]
        e2 = e2_ref[:, j * TJ:(j + 1) * TJ]
        d = lax.dot_general(x, et, (((1,), (0,)), ((), ())),
                            preferred_element_type=jnp.float32)
        dist = (x2 + e2) - 2.0 * d
        tmin = jnp.min(dist, axis=1, keepdims=True)
        iota = lax.broadcasted_iota(jnp.int32, (TI, TJ), 1)
        targ = jnp.min(jnp.where(dist == tmin, iota, TJ),
                       axis=1, keepdims=True) + j * TJ
        if mv is None:
            mv, mi = tmin, targ
        else:
            upd = tmin < mv
            mv = jnp.where(upd, tmin, mv)
            mi = jnp.where(upd, targ, mi)
    idx_ref[...] = mi


def _argmin_indices(flat, embt, x2, e2):
    grid = (N // TI,)
    return pl.pallas_call(
        _argmin_body,
        grid=grid,
        in_specs=[
            pl.BlockSpec((TI, ED), lambda i: (i, 0)),
            pl.BlockSpec((ED, NE), lambda i: (0, 0)),
            pl.BlockSpec((TI, 1), lambda i: (i, 0)),
            pl.BlockSpec((1, NE), lambda i: (0, 0)),
        ],
        out_specs=pl.BlockSpec((TI, 1), lambda i: (i, 0)),
        out_shape=jax.ShapeDtypeStruct((N, 1), jnp.int32),
    )(flat, embt, x2, e2)


_NC = 2
_NS = 16
_NW = _NC * _NS
_BPW = N // _NW


@functools.cache
def _sc_gather_kernel():
    @functools.partial(
        pl.kernel,
        mesh=plsc.VectorSubcoreMesh(core_axis_name="c", subcore_axis_name="s"),
        out_type=jax.ShapeDtypeStruct((N, ED), jnp.float32),
        scratch_types=[
            pltpu.VMEM((_BPW,), jnp.int32),
            pltpu.VMEM((_BPW, ED), jnp.float32),
            pltpu.SemaphoreType.DMA,
        ],
    )
    def _sc_gather(table_hbm, idx_hbm, out_hbm, idx_v, rows_v, sem):
        wid = lax.axis_index("s") * _NC + lax.axis_index("c")
        base = wid * _BPW
        pltpu.sync_copy(idx_hbm.at[pl.ds(base, _BPW)], idx_v)
        pltpu.async_copy(table_hbm.at[idx_v], rows_v, sem).wait()
        pltpu.sync_copy(rows_v, out_hbm.at[pl.ds(base, _BPW)])

    return _sc_gather


def kernel(inputs, embedding):
    x = jnp.transpose(inputs, (0, 2, 3, 1))
    input_shape = x.shape
    flat = x.reshape(-1, ED)
    x2 = jnp.sum(flat ** 2, axis=1, keepdims=True)
    e2 = jnp.sum(embedding ** 2, axis=1).reshape(1, NE)
    embt = embedding.T
    idx = _argmin_indices(flat, embt, x2, e2).reshape(N)
    q = _sc_gather_kernel()(embedding, idx)
    quantized = q.reshape(input_shape)
    out = x + (quantized - x)
    return jnp.transpose(out, (0, 3, 1, 2))


# speedup vs baseline: 1.3594x; 1.0816x over previous
import functools

import jax
import jax.numpy as jnp
from jax import lax
from jax.experimental import pallas as pl
from jax.experimental.pallas import tpu as pltpu
from jax.experimental.pallas import tpu_sc as plsc

NE = 8192
ED = 256
N = 8192
TJ = 512


B = 8
HW = 1024


def _argmin_body(x_ref, emb_ref, x2_ref, e2_ref, idx_ref):
    x_b = x_ref[...].reshape(ED, HW)
    x2 = x2_ref[...].reshape(1, HW)
    riota = lax.broadcasted_iota(jnp.int32, (TJ, HW), 0)
    mv = None
    mi = None
    for j in range(NE // TJ):
        et = emb_ref[j * TJ:(j + 1) * TJ, :]
        e2 = e2_ref[j * TJ:(j + 1) * TJ, :]
        d = lax.dot_general(et, x_b, (((1,), (0,)), ((), ())),
                            preferred_element_type=jnp.float32)
        dist = (x2 + e2) - 2.0 * d
        tmin = jnp.min(dist, axis=0, keepdims=True)
        targ = jnp.min(jnp.where(dist == tmin, riota, TJ),
                       axis=0, keepdims=True) + j * TJ
        if mv is None:
            mv, mi = tmin, targ
        else:
            upd = tmin < mv
            mv = jnp.where(upd, tmin, mv)
            mi = jnp.where(upd, targ, mi)
    idx_ref[...] = mi.reshape(1, 1, HW)


def _argmin_indices(x_r, embedding, x2, e2):
    return pl.pallas_call(
        _argmin_body,
        grid=(B,),
        in_specs=[
            pl.BlockSpec((1, ED, HW), lambda b: (b, 0, 0)),
            pl.BlockSpec((NE, ED), lambda b: (0, 0)),
            pl.BlockSpec((1, 1, HW), lambda b: (b, 0, 0)),
            pl.BlockSpec((NE, 1), lambda b: (0, 0)),
        ],
        out_specs=pl.BlockSpec((1, 1, HW), lambda b: (b, 0, 0)),
        out_shape=jax.ShapeDtypeStruct((B, 1, HW), jnp.int32),
    )(x_r, embedding, x2, e2)


_NC = 2
_NS = 16
_NW = _NC * _NS
_BPW = N // _NW


@functools.cache
def _sc_gather_kernel():
    @functools.partial(
        pl.kernel,
        mesh=plsc.VectorSubcoreMesh(core_axis_name="c", subcore_axis_name="s"),
        out_type=jax.ShapeDtypeStruct((N, ED), jnp.float32),
        scratch_types=[
            pltpu.VMEM((_BPW,), jnp.int32),
            pltpu.VMEM((_BPW, ED), jnp.float32),
            pltpu.SemaphoreType.DMA,
        ],
    )
    def _sc_gather(table_hbm, idx_hbm, out_hbm, idx_v, rows_v, sem):
        wid = lax.axis_index("s") * _NC + lax.axis_index("c")
        base = wid * _BPW
        pltpu.sync_copy(idx_hbm.at[pl.ds(base, _BPW)], idx_v)
        pltpu.async_copy(table_hbm.at[idx_v], rows_v, sem).wait()
        pltpu.sync_copy(rows_v, out_hbm.at[pl.ds(base, _BPW)])

    return _sc_gather


def kernel(inputs, embedding):
    x_r = inputs.reshape(B, ED, HW)
    flat = jnp.transpose(inputs, (0, 2, 3, 1)).reshape(-1, ED)
    x2 = jnp.sum(flat ** 2, axis=1, keepdims=True).reshape(B, 1, HW)
    e2 = jnp.sum(embedding ** 2, axis=1).reshape(NE, 1)
    idx = _argmin_indices(x_r, embedding, x2, e2).reshape(N)
    q = _sc_gather_kernel()(embedding, idx)
    qt = jnp.transpose(q.reshape(B, 32, 32, ED), (0, 3, 1, 2))
    return inputs + (qt - inputs)


# speedup vs baseline: 1.9274x; 1.4178x over previous
import functools

import jax
import jax.numpy as jnp
from jax import lax
from jax.experimental import pallas as pl
from jax.experimental.pallas import tpu as pltpu
from jax.experimental.pallas import tpu_sc as plsc

NE = 8192
ED = 256
N = 8192
TJ = 2048


B = 8
HW = 1024


CH = 8


def _argmin_body(x_ref, emb_ref, x2h_ref, e2h_ref, idx_ref):
    x_b = x_ref[...].reshape(ED, HW)
    x2h = x2h_ref[...].reshape(1, HW)
    acc_v = jnp.full((CH, HW), jnp.inf, jnp.float32)
    acc_c = jnp.zeros((CH, HW), jnp.int32)
    for jt in range(NE // TJ):
        et = emb_ref[jt * TJ:(jt + 1) * TJ, :]
        d = lax.dot_general(et, x_b, (((1,), (0,)), ((), ())),
                            preferred_element_type=jnp.float32)
        for a in range(TJ // CH):
            lo = a * CH
            e2h = e2h_ref[jt * TJ + lo:jt * TJ + lo + CH, :]
            dh = (x2h + e2h) - d[lo:lo + CH]
            cid = jt * (TJ // CH) + a
            lt = dh < acc_v
            acc_c = jnp.where(lt, cid, acc_c)
            acc_v = jnp.minimum(acc_v, dh)
    riota = lax.broadcasted_iota(jnp.int32, (CH, HW), 0)
    gj = acc_c * CH + riota
    m = jnp.min(acc_v, axis=0, keepdims=True)
    mi = jnp.min(jnp.where(acc_v == m, gj, NE), axis=0, keepdims=True)
    idx_ref[...] = mi.reshape(1, 1, HW)


def _argmin_indices(x_r, embedding, x2, e2):
    return pl.pallas_call(
        _argmin_body,
        grid=(B,),
        in_specs=[
            pl.BlockSpec((1, ED, HW), lambda b: (b, 0, 0)),
            pl.BlockSpec((NE, ED), lambda b: (0, 0)),
            pl.BlockSpec((1, 1, HW), lambda b: (b, 0, 0)),
            pl.BlockSpec((NE, 1), lambda b: (0, 0)),
        ],
        out_specs=pl.BlockSpec((1, 1, HW), lambda b: (b, 0, 0)),
        out_shape=jax.ShapeDtypeStruct((B, 1, HW), jnp.int32),
    )(x_r, embedding, x2, e2)


_NC = 2
_NS = 16
_NW = _NC * _NS
_BPW = N // _NW


@functools.cache
def _sc_gather_kernel():
    @functools.partial(
        pl.kernel,
        mesh=plsc.VectorSubcoreMesh(core_axis_name="c", subcore_axis_name="s"),
        out_type=jax.ShapeDtypeStruct((N, ED), jnp.float32),
        scratch_types=[
            pltpu.VMEM((_BPW,), jnp.int32),
            pltpu.VMEM((_BPW, ED), jnp.float32),
            pltpu.SemaphoreType.DMA,
        ],
    )
    def _sc_gather(table_hbm, idx_hbm, out_hbm, idx_v, rows_v, sem):
        wid = lax.axis_index("s") * _NC + lax.axis_index("c")
        base = wid * _BPW
        pltpu.sync_copy(idx_hbm.at[pl.ds(base, _BPW)], idx_v)
        pltpu.async_copy(table_hbm.at[idx_v], rows_v, sem).wait()
        pltpu.sync_copy(rows_v, out_hbm.at[pl.ds(base, _BPW)])

    return _sc_gather


def kernel(inputs, embedding):
    x_r = inputs.reshape(B, ED, HW)
    flat = jnp.transpose(inputs, (0, 2, 3, 1)).reshape(-1, ED)
    x2h = (jnp.sum(flat ** 2, axis=1, keepdims=True) * 0.5).reshape(B, 1, HW)
    e2h = (jnp.sum(embedding ** 2, axis=1) * 0.5).reshape(NE, 1)
    idx = _argmin_indices(x_r, embedding, x2h, e2h).reshape(N)
    q = _sc_gather_kernel()(embedding, idx)
    qt = jnp.transpose(q.reshape(B, 32, 32, ED), (0, 3, 1, 2))
    return inputs + (qt - inputs)


# speedup vs baseline: 1.9729x; 1.0236x over previous
import functools

import jax
import jax.numpy as jnp
from jax import lax
from jax.experimental import pallas as pl
from jax.experimental.pallas import tpu as pltpu
from jax.experimental.pallas import tpu_sc as plsc

NE = 8192
ED = 256
N = 8192
TJ = 2048


B = 8
HW = 1024


CH = 8


def _argmin_body(x_ref, emb_ref, x2h_ref, idx_ref):
    x_b = x_ref[...].reshape(ED, HW)
    x2h = x2h_ref[...].reshape(1, HW)
    acc_v = jnp.full((CH, HW), jnp.inf, jnp.float32)
    acc_c = jnp.zeros((CH, HW), jnp.int32)
    for jt in range(NE // TJ):
        et = emb_ref[jt * TJ:(jt + 1) * TJ, :]
        e2h_t = jnp.sum(et ** 2, axis=1, keepdims=True) * 0.5
        d = lax.dot_general(et, x_b, (((1,), (0,)), ((), ())),
                            preferred_element_type=jnp.float32)
        for a in range(TJ // CH):
            lo = a * CH
            e2h = e2h_t[lo:lo + CH]
            dh = (x2h + e2h) - d[lo:lo + CH]
            cid = jt * (TJ // CH) + a
            lt = dh < acc_v
            acc_c = jnp.where(lt, cid, acc_c)
            acc_v = jnp.minimum(acc_v, dh)
    riota = lax.broadcasted_iota(jnp.int32, (CH, HW), 0)
    gj = acc_c * CH + riota
    m = jnp.min(acc_v, axis=0, keepdims=True)
    mi = jnp.min(jnp.where(acc_v == m, gj, NE), axis=0, keepdims=True)
    idx_ref[...] = mi.reshape(1, 1, HW)


def _argmin_indices(x_r, embedding, x2):
    return pl.pallas_call(
        _argmin_body,
        grid=(B,),
        in_specs=[
            pl.BlockSpec((1, ED, HW), lambda b: (b, 0, 0)),
            pl.BlockSpec((NE, ED), lambda b: (0, 0)),
            pl.BlockSpec((1, 1, HW), lambda b: (b, 0, 0)),
        ],
        out_specs=pl.BlockSpec((1, 1, HW), lambda b: (b, 0, 0)),
        out_shape=jax.ShapeDtypeStruct((B, 1, HW), jnp.int32),
    )(x_r, embedding, x2)


_NC = 2
_NS = 16
_NW = _NC * _NS
_BPW = N // _NW


@functools.cache
def _sc_gather_kernel():
    @functools.partial(
        pl.kernel,
        mesh=plsc.VectorSubcoreMesh(core_axis_name="c", subcore_axis_name="s"),
        out_type=jax.ShapeDtypeStruct((N, ED), jnp.float32),
        scratch_types=[
            pltpu.VMEM((_BPW,), jnp.int32),
            pltpu.VMEM((_BPW, ED), jnp.float32),
            pltpu.SemaphoreType.DMA,
        ],
    )
    def _sc_gather(table_hbm, idx_hbm, out_hbm, idx_v, rows_v, sem):
        wid = lax.axis_index("s") * _NC + lax.axis_index("c")
        base = wid * _BPW
        pltpu.sync_copy(idx_hbm.at[pl.ds(base, _BPW)], idx_v)
        pltpu.async_copy(table_hbm.at[idx_v], rows_v, sem).wait()
        pltpu.sync_copy(rows_v, out_hbm.at[pl.ds(base, _BPW)])

    return _sc_gather


def kernel(inputs, embedding):
    x_r = inputs.reshape(B, ED, HW)
    flat = jnp.transpose(inputs, (0, 2, 3, 1)).reshape(-1, ED)
    x2h = (jnp.sum(flat ** 2, axis=1, keepdims=True) * 0.5).reshape(B, 1, HW)
    idx = _argmin_indices(x_r, embedding, x2h).reshape(N)
    q = _sc_gather_kernel()(embedding, idx)
    qt = jnp.transpose(q.reshape(B, 32, 32, ED), (0, 3, 1, 2))
    return inputs + (qt - inputs)


# speedup vs baseline: 2.1249x; 1.0771x over previous
import functools

import jax
import jax.numpy as jnp
from jax import lax
from jax.experimental import pallas as pl
from jax.experimental.pallas import tpu as pltpu
from jax.experimental.pallas import tpu_sc as plsc

NE = 8192
ED = 256
N = 8192
TJ = 2048


B = 8
HW = 1024


CH = 8


def _argmin_body(x_ref, emb_ref, idx_ref, e2h_ref):
    x_b = x_ref[...].reshape(ED, HW)
    x2h = jnp.sum(x_b ** 2, axis=0, keepdims=True) * 0.5

    @pl.when(pl.program_id(0) == 0)
    def _():
        e2h_ref[...] = jnp.sum(emb_ref[...] ** 2, axis=1, keepdims=True) * 0.5

    acc_v = jnp.full((CH, HW), jnp.inf, jnp.float32)
    acc_c = jnp.zeros((CH, HW), jnp.int32)
    for jt in range(NE // TJ):
        et = emb_ref[jt * TJ:(jt + 1) * TJ, :]
        d = lax.dot_general(et, x_b, (((1,), (0,)), ((), ())),
                            preferred_element_type=jnp.float32)
        for a in range(TJ // CH):
            lo = a * CH
            e2h = e2h_ref[jt * TJ + lo:jt * TJ + lo + CH, :]
            dh = (x2h + e2h) - d[lo:lo + CH]
            cid = jt * (TJ // CH) + a
            lt = dh < acc_v
            acc_c = jnp.where(lt, cid, acc_c)
            acc_v = jnp.minimum(acc_v, dh)
    riota = lax.broadcasted_iota(jnp.int32, (CH, HW), 0)
    gj = acc_c * CH + riota
    m = jnp.min(acc_v, axis=0, keepdims=True)
    mi = jnp.min(jnp.where(acc_v == m, gj, NE), axis=0, keepdims=True)
    idx_ref[...] = mi.reshape(1, 1, HW)


def _argmin_indices(x_r, embedding):
    return pl.pallas_call(
        _argmin_body,
        grid=(B,),
        in_specs=[
            pl.BlockSpec((1, ED, HW), lambda b: (b, 0, 0)),
            pl.BlockSpec((NE, ED), lambda b: (0, 0)),
        ],
        out_specs=pl.BlockSpec((1, 1, HW), lambda b: (b, 0, 0)),
        out_shape=jax.ShapeDtypeStruct((B, 1, HW), jnp.int32),
        scratch_shapes=[pltpu.VMEM((NE, 1), jnp.float32)],
    )(x_r, embedding)


_NC = 2
_NS = 16
_NW = _NC * _NS
_BPW = N // _NW


@functools.cache
def _sc_gather_kernel():
    @functools.partial(
        pl.kernel,
        mesh=plsc.VectorSubcoreMesh(core_axis_name="c", subcore_axis_name="s"),
        out_type=jax.ShapeDtypeStruct((N, ED), jnp.float32),
        scratch_types=[
            pltpu.VMEM((_BPW,), jnp.int32),
            pltpu.VMEM((_BPW, ED), jnp.float32),
            pltpu.SemaphoreType.DMA,
        ],
    )
    def _sc_gather(table_hbm, idx_hbm, out_hbm, idx_v, rows_v, sem):
        wid = lax.axis_index("s") * _NC + lax.axis_index("c")
        base = wid * _BPW
        pltpu.sync_copy(idx_hbm.at[pl.ds(base, _BPW)], idx_v)
        pltpu.async_copy(table_hbm.at[idx_v], rows_v, sem).wait()
        pltpu.sync_copy(rows_v, out_hbm.at[pl.ds(base, _BPW)])

    return _sc_gather


def kernel(inputs, embedding):
    x_r = inputs.reshape(B, ED, HW)
    idx = _argmin_indices(x_r, embedding).reshape(N)
    q = _sc_gather_kernel()(embedding, idx)
    qt = jnp.transpose(q.reshape(B, 32, 32, ED), (0, 3, 1, 2))
    return inputs + (qt - inputs)
